# baseline (speedup 1.0000x reference)
import functools

import jax
import jax.numpy as jnp
from jax.experimental import pallas as pl
from jax.experimental.pallas import tpu as pltpu
from jax.experimental.pallas import tpu_sc as plsc

_BIG = 9000000000000000.0
_LANES = 16
_NWORKERS = 32


_CAP = 1024


def _sc_select_kernel(sim_hbm, k_hbm, t_hbm, j_hbm, simbuf, tbuf, jbuf, kvbuf,
                      hist, candbuf, *, n):
    rows_per_w = n // _NWORKERS
    slabs = rows_per_w // _LANES
    cid = jax.lax.axis_index("c")
    sid = jax.lax.axis_index("s")
    wid = sid * 2 + cid
    base_row = wid * rows_per_w

    pltpu.sync_copy(k_hbm, kvbuf)
    kvec = kvbuf[...]
    lanes = jax.lax.iota(jnp.int32, _LANES)
    ones = jnp.ones((_LANES,), jnp.int32)
    zeros = jnp.zeros((_LANES,), jnp.int32)

    lanes_scaled = lanes * n
    neginf = jnp.full((_LANES,), -jnp.inf, jnp.float32)

    @plsc.parallel_loop(0, _CAP, unroll=8)
    def _(i):
        candbuf[pl.ds(i * _LANES, _LANES)] = zeros

    def slab_body(s, _):
        row0 = base_row + s * _LANES
        pltpu.sync_copy(sim_hbm.at[pl.ds(row0 * n, _LANES * n)], simbuf)
        rowvec = lanes + jnp.full((_LANES,), row0, jnp.int32)
        plsc.store_scatter(simbuf, [lanes_scaled + rowvec], neginf)

        def key_gather(colv):
            v = plsc.load_gather(simbuf, [lanes_scaled + colv])
            return plsc.bitcast(v, jnp.int32)

        def key_at(c):
            return key_gather(jnp.full((_LANES,), c, jnp.int32))

        def zero_hist(ncopy=1):
            @plsc.parallel_loop(0, 256 * ncopy, unroll=8)
            def _(b):
                hist[pl.ds(b * _LANES, _LANES)] = zeros

        def scan_level(k_cur, ncopy=1):
            def sbody(i, carry):
                cum, b0, above, hsel = carry
                b = 255 - i
                h = hist[pl.ds(b * _LANES, _LANES)]
                for p in range(1, ncopy):
                    h = h + hist[pl.ds((p * 256 + b) * _LANES, _LANES)]
                newcum = cum + h
                found = jnp.logical_and(cum < k_cur, newcum >= k_cur)
                bs = jnp.full((_LANES,), b, jnp.int32)
                b0 = jnp.where(found, bs, b0)
                above = jnp.where(found, cum, above)
                hsel = jnp.where(found, h, hsel)
                return newcum, b0, above, hsel

            _, b0, above, hsel = plsc.parallel_loop(
                0, 256, unroll=4, carry=(zeros, zeros, zeros, zeros))(sbody)
            return b0, above, hsel

        zero_hist(8)

        @plsc.parallel_loop(0, n, unroll=16)
        def _(c):
            idx = lanes_scaled + jnp.full((_LANES,), c, jnp.int32)
            v = plsc.load_gather(simbuf, [idx])
            xi = plsc.bitcast(v, jnp.int32)
            key = jnp.where(xi < 0, xi ^ jnp.int32(0x7FFFFFFF), xi)
            plsc.store_scatter(simbuf, [idx], plsc.bitcast(key, jnp.float32))
            bin_ = (key >> 24) + 128
            phase = jnp.full((_LANES,), (c & 7) * (256 * _LANES), jnp.int32)
            plsc.addupdate_scatter(hist, [phase + bin_ * _LANES + lanes],
                                   ones)

        b0, above, q = scan_level(kvec, 8)
        t1 = b0 - 128
        k1 = kvec - above
        qmax = jnp.max(q)

        def refine(loop_hi, col_of, valid_of, t_in, k_in, unroll):
            t = t_in
            k_cur = k_in
            for shift in (16, 8, 0):
                zero_hist()

                def lvl_body(c, shift=shift, t=t):
                    key = key_gather(col_of(c))
                    x = key >> shift
                    match = jnp.logical_and((x >> 8) == t, valid_of(c))
                    plsc.addupdate_scatter(hist,
                                           [(x & 255) * _LANES + lanes],
                                           ones, mask=match)

                plsc.parallel_loop(0, loop_hi, unroll=unroll)(lvl_body)
                b0, above, _ = scan_level(k_cur)
                t = (t << 8) | b0
                k_cur = k_cur - above

            def tie_body(c, carry):
                cnt, jcol = carry
                colv = col_of(c)
                key = key_gather(colv)
                e = jnp.logical_and(key == t, valid_of(c))
                hit = jnp.logical_and(e, cnt == k_cur - 1)
                jcol = jnp.where(hit, colv, jcol)
                cnt = cnt + jnp.where(e, 1, 0)
                return cnt, jcol

            _, jcol = plsc.parallel_loop(0, loop_hi, unroll=unroll,
                                         carry=(zeros, zeros))(tie_body)
            return t, jcol

        def fast_path():
            def comp_body(c, cnt):
                key = key_at(c)
                match = ((key >> 24) + 128) == b0
                plsc.store_scatter(candbuf, [cnt * _LANES + lanes],
                                   jnp.full((_LANES,), c, jnp.int32),
                                   mask=match)
                return cnt + jnp.where(match, 1, 0)

            plsc.parallel_loop(0, n, unroll=8, carry=zeros)(comp_body)

            def col_of(c):
                return plsc.load_gather(
                    candbuf,
                    [jnp.full((_LANES,), c * _LANES, jnp.int32) + lanes])

            def valid_of(c):
                return jnp.full((_LANES,), c, jnp.int32) < q

            return refine(qmax, col_of, valid_of, t1, k1, 4)

        def full_path():
            def col_of(c):
                return jnp.full((_LANES,), c, jnp.int32)

            def valid_of(c):
                return jnp.full((_LANES,), True, jnp.bool_)

            return refine(n, col_of, valid_of, t1, k1, 8)

        t, jcol = jax.lax.cond(qmax <= _CAP, fast_path, full_path)

        out_idx = lanes + jnp.full((_LANES,), s * _LANES, jnp.int32)
        plsc.store_scatter(tbuf, [out_idx], t)
        plsc.store_scatter(jbuf, [out_idx], jcol)
        return 0

    jax.lax.fori_loop(0, slabs, slab_body, 0)
    pltpu.sync_copy(tbuf, t_hbm.at[pl.ds(base_row, rows_per_w)])
    pltpu.sync_copy(jbuf, j_hbm.at[pl.ds(base_row, rows_per_w)])


def _centrality_kernel(mb_ref, out_ref):
    s = jnp.sum(mb_ref[...], axis=-1) / mb_ref.shape[-1]
    out_ref[...] = s.reshape(1, 1, -1)


def _loss_kernel(sim_ref, cent_ref, temp_ref, t_ref, j_ref, out_ref, *, rblk,
                 n):
    i = pl.program_id(0)
    sim = sim_ref[...]
    cent = cent_ref[...]
    temp = temp_ref[0, 0]
    tk = t_ref[...]
    jsel = j_ref[...]
    big = jnp.float32(_BIG)

    rows = i * rblk + jax.lax.broadcasted_iota(jnp.int32, (rblk, n), 0)
    cols = jax.lax.broadcasted_iota(jnp.int32, (rblk, n), 1)
    diag = rows == cols

    xi = jax.lax.bitcast_convert_type(sim, jnp.int32)
    key = jnp.where(xi < 0, xi ^ jnp.int32(0x7FFFFFFF), xi)
    key = jnp.where(diag, jnp.int32(-(2**31)), key)

    gt = key > tk
    eq = jnp.logical_and(key == tk, jnp.logical_not(diag))
    neighbor = jnp.logical_or(gt, jnp.logical_and(eq, cols <= jsel))
    extended = jnp.logical_or(neighbor, diag)
    comp = jnp.logical_not(extended)

    min_s = jnp.min(jnp.where(comp, sim, big), axis=1, keepdims=True)
    max_s = jnp.max(jnp.where(comp, sim, -big), axis=1, keepdims=True)
    min_c = jnp.min(jnp.where(comp, cent, big), axis=1, keepdims=True)
    max_c = jnp.max(jnp.where(comp, cent, -big), axis=1, keepdims=True)

    norm_s = (sim - min_s) / (max_s - min_s)
    norm_c = (cent - min_c) / (max_c - min_c)
    adj = jnp.where(neighbor, norm_s - norm_c, -big) * temp

    amax = jnp.max(adj, axis=1, keepdims=True)
    e = jnp.exp(adj - amax)
    w = e / jnp.sum(e, axis=1, keepdims=True)
    w = jnp.where(neighbor, w, 0.0)
    w = jnp.where(diag, 1.0, w)

    msim = jnp.where(extended, sim, -big)
    lmax = jnp.max(msim, axis=1, keepdims=True)
    lse = lmax + jnp.log(jnp.sum(jnp.exp(msim - lmax), axis=1, keepdims=True))
    lp = msim - lse

    numer = jnp.sum(w * lp, axis=1)
    denom = jnp.sum(w, axis=1)
    row_loss = -numer / denom

    @pl.when(i == 0)
    def _():
        out_ref[...] = jnp.zeros_like(out_ref)

    out_ref[...] += (jnp.sum(row_loss) / n).reshape(1, 1)


def kernel(similarity_matrix, memory_bank_matrix, num_neighbors, temperature):
    n = similarity_matrix.shape[0]
    rows_per_w = n // _NWORKERS

    karr = jnp.full((_LANES,), num_neighbors, jnp.int32)
    mesh = plsc.VectorSubcoreMesh(core_axis_name="c", subcore_axis_name="s")
    tsel, jsel = pl.kernel(
        functools.partial(_sc_select_kernel, n=n),
        mesh=mesh,
        compiler_params=pltpu.CompilerParams(needs_layout_passes=False),
        out_type=[
            jax.ShapeDtypeStruct((n,), jnp.int32),
            jax.ShapeDtypeStruct((n,), jnp.int32),
        ],
        scratch_types=[
            pltpu.VMEM((_LANES * n,), jnp.float32),
            pltpu.VMEM((rows_per_w,), jnp.int32),
            pltpu.VMEM((rows_per_w,), jnp.int32),
            pltpu.VMEM((_LANES,), jnp.int32),
            pltpu.VMEM((8 * 256 * _LANES,), jnp.int32),
            pltpu.VMEM((_CAP * _LANES,), jnp.int32),
        ],
    )(similarity_matrix.reshape(-1), karr)

    cblk = 256
    cent = pl.pallas_call(
        _centrality_kernel,
        grid=(n // cblk,),
        in_specs=[pl.BlockSpec((cblk, n), lambda i: (i, 0))],
        out_specs=pl.BlockSpec((1, 1, cblk), lambda i: (i, 0, 0)),
        out_shape=jax.ShapeDtypeStruct((n // cblk, 1, cblk), jnp.float32),
    )(memory_bank_matrix)
    cent = cent.reshape(1, n)

    rblk = 256
    loss = pl.pallas_call(
        functools.partial(_loss_kernel, rblk=rblk, n=n),
        grid=(n // rblk,),
        in_specs=[
            pl.BlockSpec((rblk, n), lambda i: (i, 0)),
            pl.BlockSpec((1, n), lambda i: (0, 0)),
            pl.BlockSpec((1, 1), lambda i: (0, 0)),
            pl.BlockSpec((rblk, 1), lambda i: (i, 0)),
            pl.BlockSpec((rblk, 1), lambda i: (i, 0)),
        ],
        out_specs=pl.BlockSpec((1, 1), lambda i: (0, 0)),
        out_shape=jax.ShapeDtypeStruct((1, 1), jnp.float32),
    )(
        similarity_matrix,
        cent,
        jnp.asarray(temperature, jnp.float32).reshape(1, 1),
        tsel.reshape(n, 1),
        jsel.reshape(n, 1),
    )
    return loss[0, 0]


# speedup vs baseline: 1.0220x; 1.0220x over previous
import functools

import jax
import jax.numpy as jnp
from jax.experimental import pallas as pl
from jax.experimental.pallas import tpu as pltpu
from jax.experimental.pallas import tpu_sc as plsc

_BIG = 9000000000000000.0
_LANES = 16
_NWORKERS = 32


_CAP = 1024


def _sc_select_kernel(sim_hbm, k_hbm, t_hbm, j_hbm, simbuf, tbuf, jbuf, kvbuf,
                      hist, candbuf, *, n):
    rows_per_w = n // _NWORKERS
    slabs = rows_per_w // _LANES
    cid = jax.lax.axis_index("c")
    sid = jax.lax.axis_index("s")
    wid = sid * 2 + cid
    base_row = wid * rows_per_w

    pltpu.sync_copy(k_hbm, kvbuf)
    kvec = kvbuf[...]
    lanes = jax.lax.iota(jnp.int32, _LANES)
    ones = jnp.ones((_LANES,), jnp.int32)
    zeros = jnp.zeros((_LANES,), jnp.int32)

    lanes_scaled = lanes * n
    neginf = jnp.full((_LANES,), -jnp.inf, jnp.float32)

    @plsc.parallel_loop(0, _CAP, unroll=8)
    def _(i):
        candbuf[pl.ds(i * _LANES, _LANES)] = zeros

    def slab_body(s, _):
        row0 = base_row + s * _LANES
        pltpu.sync_copy(sim_hbm.at[pl.ds(row0 * n, _LANES * n)], simbuf)
        rowvec = lanes + jnp.full((_LANES,), row0, jnp.int32)
        plsc.store_scatter(simbuf, [lanes_scaled + rowvec], neginf)

        def key_gather(colv):
            v = plsc.load_gather(simbuf, [lanes_scaled + colv])
            return plsc.bitcast(v, jnp.int32)

        def key_at(c):
            return key_gather(jnp.full((_LANES,), c, jnp.int32))

        def zero_hist(ncopy=1):
            @plsc.parallel_loop(0, 256 * ncopy, unroll=8)
            def _(b):
                hist[pl.ds(b * _LANES, _LANES)] = zeros

        def scan_level(k_cur, ncopy=1):
            def sbody(i, carry):
                cum, b0, above, hsel = carry
                b = 255 - i
                h = hist[pl.ds(b * _LANES, _LANES)]
                for p in range(1, ncopy):
                    h = h + hist[pl.ds((p * 256 + b) * _LANES, _LANES)]
                newcum = cum + h
                found = jnp.logical_and(cum < k_cur, newcum >= k_cur)
                bs = jnp.full((_LANES,), b, jnp.int32)
                b0 = jnp.where(found, bs, b0)
                above = jnp.where(found, cum, above)
                hsel = jnp.where(found, h, hsel)
                return newcum, b0, above, hsel

            _, b0, above, hsel = plsc.parallel_loop(
                0, 256, unroll=4, carry=(zeros, zeros, zeros, zeros))(sbody)
            return b0, above, hsel

        zero_hist()

        @plsc.parallel_loop(0, n, unroll=16)
        def _(c):
            idx = lanes_scaled + jnp.full((_LANES,), c, jnp.int32)
            v = plsc.load_gather(simbuf, [idx])
            xi = plsc.bitcast(v, jnp.int32)
            key = jnp.where(xi < 0, xi ^ jnp.int32(0x7FFFFFFF), xi)
            plsc.store_scatter(simbuf, [idx], plsc.bitcast(key, jnp.float32))
            bin_ = (key >> 24) + 128
            plsc.addupdate_scatter(hist, [bin_ * _LANES + lanes], ones)

        b0, above, q = scan_level(kvec)
        t1 = b0 - 128
        k1 = kvec - above
        qmax = jnp.max(q)

        def refine(loop_hi, col_of, valid_of, t_in, k_in, unroll):
            t = t_in
            k_cur = k_in
            for shift in (16, 8, 0):
                zero_hist()

                def lvl_body(c, shift=shift, t=t):
                    key = key_gather(col_of(c))
                    x = key >> shift
                    match = jnp.logical_and((x >> 8) == t, valid_of(c))
                    plsc.addupdate_scatter(hist,
                                           [(x & 255) * _LANES + lanes],
                                           ones, mask=match)

                plsc.parallel_loop(0, loop_hi, unroll=unroll)(lvl_body)
                b0, above, _ = scan_level(k_cur)
                t = (t << 8) | b0
                k_cur = k_cur - above

            def tie_body(c, carry):
                cnt, jcol = carry
                colv = col_of(c)
                key = key_gather(colv)
                e = jnp.logical_and(key == t, valid_of(c))
                hit = jnp.logical_and(e, cnt == k_cur - 1)
                jcol = jnp.where(hit, colv, jcol)
                cnt = cnt + jnp.where(e, 1, 0)
                return cnt, jcol

            _, jcol = plsc.parallel_loop(0, loop_hi, unroll=unroll,
                                         carry=(zeros, zeros))(tie_body)
            return t, jcol

        def fast_path():
            def comp_body(c, cnt):
                key = key_at(c)
                match = ((key >> 24) + 128) == b0
                plsc.store_scatter(candbuf, [cnt * _LANES + lanes],
                                   jnp.full((_LANES,), c, jnp.int32),
                                   mask=match)
                return cnt + jnp.where(match, 1, 0)

            plsc.parallel_loop(0, n, unroll=16, carry=zeros)(comp_body)

            def col_of(c):
                return plsc.load_gather(
                    candbuf,
                    [jnp.full((_LANES,), c * _LANES, jnp.int32) + lanes])

            def valid_of(c):
                return jnp.full((_LANES,), c, jnp.int32) < q

            return refine(qmax, col_of, valid_of, t1, k1, 4)

        def full_path():
            def col_of(c):
                return jnp.full((_LANES,), c, jnp.int32)

            def valid_of(c):
                return jnp.full((_LANES,), True, jnp.bool_)

            return refine(n, col_of, valid_of, t1, k1, 8)

        t, jcol = jax.lax.cond(qmax <= _CAP, fast_path, full_path)

        out_idx = lanes + jnp.full((_LANES,), s * _LANES, jnp.int32)
        plsc.store_scatter(tbuf, [out_idx], t)
        plsc.store_scatter(jbuf, [out_idx], jcol)
        return 0

    jax.lax.fori_loop(0, slabs, slab_body, 0)
    pltpu.sync_copy(tbuf, t_hbm.at[pl.ds(base_row, rows_per_w)])
    pltpu.sync_copy(jbuf, j_hbm.at[pl.ds(base_row, rows_per_w)])


def _centrality_kernel(mb_ref, out_ref):
    s = jnp.sum(mb_ref[...], axis=-1) / mb_ref.shape[-1]
    out_ref[...] = s.reshape(1, 1, -1)


def _loss_kernel(sim_ref, cent_ref, temp_ref, t_ref, j_ref, out_ref, *, rblk,
                 n):
    i = pl.program_id(0)
    sim = sim_ref[...]
    cent = cent_ref[...]
    temp = temp_ref[0, 0]
    tk = t_ref[...]
    jsel = j_ref[...]
    big = jnp.float32(_BIG)

    rows = i * rblk + jax.lax.broadcasted_iota(jnp.int32, (rblk, n), 0)
    cols = jax.lax.broadcasted_iota(jnp.int32, (rblk, n), 1)
    diag = rows == cols

    xi = jax.lax.bitcast_convert_type(sim, jnp.int32)
    key = jnp.where(xi < 0, xi ^ jnp.int32(0x7FFFFFFF), xi)
    key = jnp.where(diag, jnp.int32(-(2**31)), key)

    gt = key > tk
    eq = jnp.logical_and(key == tk, jnp.logical_not(diag))
    neighbor = jnp.logical_or(gt, jnp.logical_and(eq, cols <= jsel))
    extended = jnp.logical_or(neighbor, diag)
    comp = jnp.logical_not(extended)

    min_s = jnp.min(jnp.where(comp, sim, big), axis=1, keepdims=True)
    max_s = jnp.max(jnp.where(comp, sim, -big), axis=1, keepdims=True)
    min_c = jnp.min(jnp.where(comp, cent, big), axis=1, keepdims=True)
    max_c = jnp.max(jnp.where(comp, cent, -big), axis=1, keepdims=True)

    norm_s = (sim - min_s) / (max_s - min_s)
    norm_c = (cent - min_c) / (max_c - min_c)
    adj = jnp.where(neighbor, norm_s - norm_c, -big) * temp

    amax = jnp.max(adj, axis=1, keepdims=True)
    e = jnp.exp(adj - amax)
    w = e / jnp.sum(e, axis=1, keepdims=True)
    w = jnp.where(neighbor, w, 0.0)
    w = jnp.where(diag, 1.0, w)

    msim = jnp.where(extended, sim, -big)
    lmax = jnp.max(msim, axis=1, keepdims=True)
    lse = lmax + jnp.log(jnp.sum(jnp.exp(msim - lmax), axis=1, keepdims=True))
    lp = msim - lse

    numer = jnp.sum(w * lp, axis=1)
    denom = jnp.sum(w, axis=1)
    row_loss = -numer / denom

    @pl.when(i == 0)
    def _():
        out_ref[...] = jnp.zeros_like(out_ref)

    out_ref[...] += (jnp.sum(row_loss) / n).reshape(1, 1)


def kernel(similarity_matrix, memory_bank_matrix, num_neighbors, temperature):
    n = similarity_matrix.shape[0]
    rows_per_w = n // _NWORKERS

    karr = jnp.full((_LANES,), num_neighbors, jnp.int32)
    mesh = plsc.VectorSubcoreMesh(core_axis_name="c", subcore_axis_name="s")
    tsel, jsel = pl.kernel(
        functools.partial(_sc_select_kernel, n=n),
        mesh=mesh,
        compiler_params=pltpu.CompilerParams(needs_layout_passes=False),
        out_type=[
            jax.ShapeDtypeStruct((n,), jnp.int32),
            jax.ShapeDtypeStruct((n,), jnp.int32),
        ],
        scratch_types=[
            pltpu.VMEM((_LANES * n,), jnp.float32),
            pltpu.VMEM((rows_per_w,), jnp.int32),
            pltpu.VMEM((rows_per_w,), jnp.int32),
            pltpu.VMEM((_LANES,), jnp.int32),
            pltpu.VMEM((8 * 256 * _LANES,), jnp.int32),
            pltpu.VMEM((_CAP * _LANES,), jnp.int32),
        ],
    )(similarity_matrix.reshape(-1), karr)

    cblk = 256
    cent = pl.pallas_call(
        _centrality_kernel,
        grid=(n // cblk,),
        in_specs=[pl.BlockSpec((cblk, n), lambda i: (i, 0))],
        out_specs=pl.BlockSpec((1, 1, cblk), lambda i: (i, 0, 0)),
        out_shape=jax.ShapeDtypeStruct((n // cblk, 1, cblk), jnp.float32),
    )(memory_bank_matrix)
    cent = cent.reshape(1, n)

    rblk = 256
    loss = pl.pallas_call(
        functools.partial(_loss_kernel, rblk=rblk, n=n),
        grid=(n // rblk,),
        in_specs=[
            pl.BlockSpec((rblk, n), lambda i: (i, 0)),
            pl.BlockSpec((1, n), lambda i: (0, 0)),
            pl.BlockSpec((1, 1), lambda i: (0, 0)),
            pl.BlockSpec((rblk, 1), lambda i: (i, 0)),
            pl.BlockSpec((rblk, 1), lambda i: (i, 0)),
        ],
        out_specs=pl.BlockSpec((1, 1), lambda i: (0, 0)),
        out_shape=jax.ShapeDtypeStruct((1, 1), jnp.float32),
    )(
        similarity_matrix,
        cent,
        jnp.asarray(temperature, jnp.float32).reshape(1, 1),
        tsel.reshape(n, 1),
        jsel.reshape(n, 1),
    )
    return loss[0, 0]


# speedup vs baseline: 1.1519x; 1.1271x over previous
import functools

import jax
import jax.numpy as jnp
from jax.experimental import pallas as pl
from jax.experimental.pallas import tpu as pltpu
from jax.experimental.pallas import tpu_sc as plsc

_BIG = 9000000000000000.0
_LANES = 16
_NWORKERS = 32


_CAP = 1024


def _sc_select_kernel(sim_hbm, k_hbm, t_hbm, j_hbm, simbuf, tbuf, jbuf, kvbuf,
                      hist, candbuf, *, n):
    rows_per_w = n // _NWORKERS
    slabs = rows_per_w // _LANES
    cid = jax.lax.axis_index("c")
    sid = jax.lax.axis_index("s")
    wid = sid * 2 + cid
    base_row = wid * rows_per_w

    pltpu.sync_copy(k_hbm, kvbuf)
    kvec = kvbuf[...]
    lanes = jax.lax.iota(jnp.int32, _LANES)
    ones = jnp.ones((_LANES,), jnp.int32)
    zeros = jnp.zeros((_LANES,), jnp.int32)

    lanes_scaled = lanes * n
    neginf = jnp.full((_LANES,), -jnp.inf, jnp.float32)

    @plsc.parallel_loop(0, _CAP, unroll=8)
    def _(i):
        candbuf[pl.ds(i * _LANES, _LANES)] = zeros

    def slab_body(s, b_pred):
        row0 = base_row + s * _LANES
        pltpu.sync_copy(sim_hbm.at[pl.ds(row0 * n, _LANES * n)], simbuf)
        rowvec = lanes + jnp.full((_LANES,), row0, jnp.int32)
        plsc.store_scatter(simbuf, [lanes_scaled + rowvec], neginf)

        def key_gather(colv):
            v = plsc.load_gather(simbuf, [lanes_scaled + colv])
            return plsc.bitcast(v, jnp.int32)

        def key_at(c):
            return key_gather(jnp.full((_LANES,), c, jnp.int32))

        def zero_hist(ncopy=1):
            @plsc.parallel_loop(0, 256 * ncopy, unroll=8)
            def _(b):
                hist[pl.ds(b * _LANES, _LANES)] = zeros

        def scan_level(k_cur, ncopy=1):
            def sbody(i, carry):
                cum, b0, above, hsel = carry
                b = 255 - i
                h = hist[pl.ds(b * _LANES, _LANES)]
                for p in range(1, ncopy):
                    h = h + hist[pl.ds((p * 256 + b) * _LANES, _LANES)]
                newcum = cum + h
                found = jnp.logical_and(cum < k_cur, newcum >= k_cur)
                bs = jnp.full((_LANES,), b, jnp.int32)
                b0 = jnp.where(found, bs, b0)
                above = jnp.where(found, cum, above)
                hsel = jnp.where(found, h, hsel)
                return newcum, b0, above, hsel

            _, b0, above, hsel = plsc.parallel_loop(
                0, 256, unroll=4, carry=(zeros, zeros, zeros, zeros))(sbody)
            return b0, above, hsel

        zero_hist()
        capv = jnp.full((_LANES,), _CAP, jnp.int32)

        def p1_body(c, cnt):
            idx = lanes_scaled + jnp.full((_LANES,), c, jnp.int32)
            v = plsc.load_gather(simbuf, [idx])
            xi = plsc.bitcast(v, jnp.int32)
            key = jnp.where(xi < 0, xi ^ jnp.int32(0x7FFFFFFF), xi)
            plsc.store_scatter(simbuf, [idx], plsc.bitcast(key, jnp.float32))
            bin_ = (key >> 24) + 128
            plsc.addupdate_scatter(hist, [bin_ * _LANES + lanes], ones)
            am = jnp.logical_and(bin_ == b_pred, cnt < capv)
            plsc.store_scatter(candbuf, [cnt * _LANES + lanes],
                               jnp.full((_LANES,), c, jnp.int32), mask=am)
            return cnt + jnp.where(am, 1, 0)

        plsc.parallel_loop(0, n, unroll=16, carry=zeros)(p1_body)

        b0, above, q = scan_level(kvec)
        t1 = b0 - 128
        k1 = kvec - above
        qmax = jnp.max(q)
        pred_ok = jnp.all(b0 == b_pred)

        def refine(loop_hi, col_of, valid_of, t_in, k_in, unroll):
            t = t_in
            k_cur = k_in
            for shift in (16, 8, 0):
                zero_hist()

                def lvl_body(c, shift=shift, t=t):
                    key = key_gather(col_of(c))
                    x = key >> shift
                    match = jnp.logical_and((x >> 8) == t, valid_of(c))
                    plsc.addupdate_scatter(hist,
                                           [(x & 255) * _LANES + lanes],
                                           ones, mask=match)

                plsc.parallel_loop(0, loop_hi, unroll=unroll)(lvl_body)
                b0, above, _ = scan_level(k_cur)
                t = (t << 8) | b0
                k_cur = k_cur - above

            def tie_body(c, carry):
                cnt, jcol = carry
                colv = col_of(c)
                key = key_gather(colv)
                e = jnp.logical_and(key == t, valid_of(c))
                hit = jnp.logical_and(e, cnt == k_cur - 1)
                jcol = jnp.where(hit, colv, jcol)
                cnt = cnt + jnp.where(e, 1, 0)
                return cnt, jcol

            _, jcol = plsc.parallel_loop(0, loop_hi, unroll=unroll,
                                         carry=(zeros, zeros))(tie_body)
            return t, jcol

        def fast_path():
            def rebuild():
                def comp_body(c, cnt):
                    key = key_at(c)
                    match = ((key >> 24) + 128) == b0
                    plsc.store_scatter(candbuf, [cnt * _LANES + lanes],
                                       jnp.full((_LANES,), c, jnp.int32),
                                       mask=match)
                    return cnt + jnp.where(match, 1, 0)

                plsc.parallel_loop(0, n, unroll=16, carry=zeros)(comp_body)
                return jnp.int32(0)

            jax.lax.cond(pred_ok, lambda: jnp.int32(0), rebuild)

            def col_of(c):
                return plsc.load_gather(
                    candbuf,
                    [jnp.full((_LANES,), c * _LANES, jnp.int32) + lanes])

            def valid_of(c):
                return jnp.full((_LANES,), c, jnp.int32) < q

            return refine(qmax, col_of, valid_of, t1, k1, 4)

        def full_path():
            def col_of(c):
                return jnp.full((_LANES,), c, jnp.int32)

            def valid_of(c):
                return jnp.full((_LANES,), True, jnp.bool_)

            return refine(n, col_of, valid_of, t1, k1, 8)

        t, jcol = jax.lax.cond(qmax <= _CAP, fast_path, full_path)

        out_idx = lanes + jnp.full((_LANES,), s * _LANES, jnp.int32)
        plsc.store_scatter(tbuf, [out_idx], t)
        plsc.store_scatter(jbuf, [out_idx], jcol)
        return b0

    jax.lax.fori_loop(0, slabs, slab_body,
                      jnp.full((_LANES,), -1, jnp.int32))
    pltpu.sync_copy(tbuf, t_hbm.at[pl.ds(base_row, rows_per_w)])
    pltpu.sync_copy(jbuf, j_hbm.at[pl.ds(base_row, rows_per_w)])


def _centrality_kernel(mb_ref, out_ref):
    s = jnp.sum(mb_ref[...], axis=-1) / mb_ref.shape[-1]
    out_ref[...] = s.reshape(1, 1, -1)


def _loss_kernel(sim_ref, cent_ref, temp_ref, t_ref, j_ref, out_ref, *, rblk,
                 n):
    i = pl.program_id(0)
    sim = sim_ref[...]
    cent = cent_ref[...]
    temp = temp_ref[0, 0]
    tk = t_ref[...]
    jsel = j_ref[...]
    big = jnp.float32(_BIG)

    rows = i * rblk + jax.lax.broadcasted_iota(jnp.int32, (rblk, n), 0)
    cols = jax.lax.broadcasted_iota(jnp.int32, (rblk, n), 1)
    diag = rows == cols

    xi = jax.lax.bitcast_convert_type(sim, jnp.int32)
    key = jnp.where(xi < 0, xi ^ jnp.int32(0x7FFFFFFF), xi)
    key = jnp.where(diag, jnp.int32(-(2**31)), key)

    gt = key > tk
    eq = jnp.logical_and(key == tk, jnp.logical_not(diag))
    neighbor = jnp.logical_or(gt, jnp.logical_and(eq, cols <= jsel))
    extended = jnp.logical_or(neighbor, diag)
    comp = jnp.logical_not(extended)

    min_s = jnp.min(jnp.where(comp, sim, big), axis=1, keepdims=True)
    max_s = jnp.max(jnp.where(comp, sim, -big), axis=1, keepdims=True)
    min_c = jnp.min(jnp.where(comp, cent, big), axis=1, keepdims=True)
    max_c = jnp.max(jnp.where(comp, cent, -big), axis=1, keepdims=True)

    norm_s = (sim - min_s) / (max_s - min_s)
    norm_c = (cent - min_c) / (max_c - min_c)
    adj = jnp.where(neighbor, norm_s - norm_c, -big) * temp

    amax = jnp.max(adj, axis=1, keepdims=True)
    e = jnp.exp(adj - amax)
    w = e / jnp.sum(e, axis=1, keepdims=True)
    w = jnp.where(neighbor, w, 0.0)
    w = jnp.where(diag, 1.0, w)

    msim = jnp.where(extended, sim, -big)
    lmax = jnp.max(msim, axis=1, keepdims=True)
    lse = lmax + jnp.log(jnp.sum(jnp.exp(msim - lmax), axis=1, keepdims=True))
    lp = msim - lse

    numer = jnp.sum(w * lp, axis=1)
    denom = jnp.sum(w, axis=1)
    row_loss = -numer / denom

    @pl.when(i == 0)
    def _():
        out_ref[...] = jnp.zeros_like(out_ref)

    out_ref[...] += (jnp.sum(row_loss) / n).reshape(1, 1)


def kernel(similarity_matrix, memory_bank_matrix, num_neighbors, temperature):
    n = similarity_matrix.shape[0]
    rows_per_w = n // _NWORKERS

    karr = jnp.full((_LANES,), num_neighbors, jnp.int32)
    mesh = plsc.VectorSubcoreMesh(core_axis_name="c", subcore_axis_name="s")
    tsel, jsel = pl.kernel(
        functools.partial(_sc_select_kernel, n=n),
        mesh=mesh,
        compiler_params=pltpu.CompilerParams(needs_layout_passes=False),
        out_type=[
            jax.ShapeDtypeStruct((n,), jnp.int32),
            jax.ShapeDtypeStruct((n,), jnp.int32),
        ],
        scratch_types=[
            pltpu.VMEM((_LANES * n,), jnp.float32),
            pltpu.VMEM((rows_per_w,), jnp.int32),
            pltpu.VMEM((rows_per_w,), jnp.int32),
            pltpu.VMEM((_LANES,), jnp.int32),
            pltpu.VMEM((8 * 256 * _LANES,), jnp.int32),
            pltpu.VMEM((_CAP * _LANES,), jnp.int32),
        ],
    )(similarity_matrix.reshape(-1), karr)

    cblk = 256
    cent = pl.pallas_call(
        _centrality_kernel,
        grid=(n // cblk,),
        in_specs=[pl.BlockSpec((cblk, n), lambda i: (i, 0))],
        out_specs=pl.BlockSpec((1, 1, cblk), lambda i: (i, 0, 0)),
        out_shape=jax.ShapeDtypeStruct((n // cblk, 1, cblk), jnp.float32),
    )(memory_bank_matrix)
    cent = cent.reshape(1, n)

    rblk = 256
    loss = pl.pallas_call(
        functools.partial(_loss_kernel, rblk=rblk, n=n),
        grid=(n // rblk,),
        in_specs=[
            pl.BlockSpec((rblk, n), lambda i: (i, 0)),
            pl.BlockSpec((1, n), lambda i: (0, 0)),
            pl.BlockSpec((1, 1), lambda i: (0, 0)),
            pl.BlockSpec((rblk, 1), lambda i: (i, 0)),
            pl.BlockSpec((rblk, 1), lambda i: (i, 0)),
        ],
        out_specs=pl.BlockSpec((1, 1), lambda i: (0, 0)),
        out_shape=jax.ShapeDtypeStruct((1, 1), jnp.float32),
    )(
        similarity_matrix,
        cent,
        jnp.asarray(temperature, jnp.float32).reshape(1, 1),
        tsel.reshape(n, 1),
        jsel.reshape(n, 1),
    )
    return loss[0, 0]


# speedup vs baseline: 1.3869x; 1.2040x over previous
import functools

import jax
import jax.numpy as jnp
from jax.experimental import pallas as pl
from jax.experimental.pallas import tpu as pltpu
from jax.experimental.pallas import tpu_sc as plsc

_BIG = 9000000000000000.0
_LANES = 16
_NWORKERS = 32


_CAP = 1024


def _sc_select_kernel(sim_hbm, k_hbm, t_hbm, j_hbm, simbuf, tbuf, jbuf, kvbuf,
                      hist, candbuf, *, n):
    rows_per_w = n // _NWORKERS
    slabs = rows_per_w // _LANES
    cid = jax.lax.axis_index("c")
    sid = jax.lax.axis_index("s")
    wid = sid * 2 + cid
    base_row = wid * rows_per_w

    pltpu.sync_copy(k_hbm, kvbuf)
    kvec = kvbuf[...]
    lanes = jax.lax.iota(jnp.int32, _LANES)
    ones = jnp.ones((_LANES,), jnp.int32)
    zeros = jnp.zeros((_LANES,), jnp.int32)

    lanes_scaled = lanes * n
    neginf = jnp.full((_LANES,), -jnp.inf, jnp.float32)

    @plsc.parallel_loop(0, _CAP, unroll=8)
    def _(i):
        candbuf[pl.ds(i * _LANES, _LANES)] = zeros

    def slab_body(s, b_pred):
        row0 = base_row + s * _LANES
        pltpu.sync_copy(sim_hbm.at[pl.ds(row0 * n, _LANES * n)], simbuf)
        rowvec = lanes + jnp.full((_LANES,), row0, jnp.int32)
        plsc.store_scatter(simbuf, [lanes_scaled + rowvec], neginf)

        def key_gather(colv):
            v = plsc.load_gather(simbuf, [lanes_scaled + colv])
            xi = plsc.bitcast(v, jnp.int32)
            return jnp.where(xi < 0, xi ^ jnp.int32(0x7FFFFFFF), xi)

        def key_at(c):
            return key_gather(jnp.full((_LANES,), c, jnp.int32))

        def zero_hist(ncopy=1):
            @plsc.parallel_loop(0, 256 * ncopy, unroll=8)
            def _(b):
                hist[pl.ds(b * _LANES, _LANES)] = zeros

        def scan_level(k_cur, ncopy=1):
            def sbody(i, carry):
                cum, b0, above, hsel = carry
                b = 255 - i
                h = hist[pl.ds(b * _LANES, _LANES)]
                for p in range(1, ncopy):
                    h = h + hist[pl.ds((p * 256 + b) * _LANES, _LANES)]
                newcum = cum + h
                found = jnp.logical_and(cum < k_cur, newcum >= k_cur)
                bs = jnp.full((_LANES,), b, jnp.int32)
                b0 = jnp.where(found, bs, b0)
                above = jnp.where(found, cum, above)
                hsel = jnp.where(found, h, hsel)
                return newcum, b0, above, hsel

            _, b0, above, hsel = plsc.parallel_loop(
                0, 256, unroll=4, carry=(zeros, zeros, zeros, zeros))(sbody)
            return b0, above, hsel

        zero_hist()
        capv = jnp.full((_LANES,), _CAP, jnp.int32)

        def p1_body(c, cnt):
            key = key_at(c)
            bin_ = (key >> 24) + 128
            plsc.addupdate_scatter(hist, [bin_ * _LANES + lanes], ones)
            am = jnp.logical_and(bin_ == b_pred, cnt < capv)
            plsc.store_scatter(candbuf, [cnt * _LANES + lanes],
                               jnp.full((_LANES,), c, jnp.int32), mask=am)
            return cnt + jnp.where(am, 1, 0)

        plsc.parallel_loop(0, n, unroll=16, carry=zeros)(p1_body)

        b0, above, q = scan_level(kvec)
        t1 = b0 - 128
        k1 = kvec - above
        qmax = jnp.max(q)
        pred_ok = jnp.all(b0 == b_pred)

        def refine(loop_hi, col_of, valid_of, t_in, k_in, unroll):
            t = t_in
            k_cur = k_in
            for shift in (16, 8, 0):
                zero_hist()

                def lvl_body(c, shift=shift, t=t):
                    key = key_gather(col_of(c))
                    x = key >> shift
                    match = jnp.logical_and((x >> 8) == t, valid_of(c))
                    plsc.addupdate_scatter(hist,
                                           [(x & 255) * _LANES + lanes],
                                           ones, mask=match)

                plsc.parallel_loop(0, loop_hi, unroll=unroll)(lvl_body)
                b0, above, _ = scan_level(k_cur)
                t = (t << 8) | b0
                k_cur = k_cur - above

            def tie_body(c, carry):
                cnt, jcol = carry
                colv = col_of(c)
                key = key_gather(colv)
                e = jnp.logical_and(key == t, valid_of(c))
                hit = jnp.logical_and(e, cnt == k_cur - 1)
                jcol = jnp.where(hit, colv, jcol)
                cnt = cnt + jnp.where(e, 1, 0)
                return cnt, jcol

            _, jcol = plsc.parallel_loop(0, loop_hi, unroll=unroll,
                                         carry=(zeros, zeros))(tie_body)
            return t, jcol

        def fast_path():
            def rebuild():
                def comp_body(c, cnt):
                    key = key_at(c)
                    match = ((key >> 24) + 128) == b0
                    plsc.store_scatter(candbuf, [cnt * _LANES + lanes],
                                       jnp.full((_LANES,), c, jnp.int32),
                                       mask=match)
                    return cnt + jnp.where(match, 1, 0)

                plsc.parallel_loop(0, n, unroll=16, carry=zeros)(comp_body)
                return jnp.int32(0)

            jax.lax.cond(pred_ok, lambda: jnp.int32(0), rebuild)

            def col_of(c):
                return plsc.load_gather(
                    candbuf,
                    [jnp.full((_LANES,), c * _LANES, jnp.int32) + lanes])

            def valid_of(c):
                return jnp.full((_LANES,), c, jnp.int32) < q

            return refine(qmax, col_of, valid_of, t1, k1, 4)

        def full_path():
            def col_of(c):
                return jnp.full((_LANES,), c, jnp.int32)

            def valid_of(c):
                return jnp.full((_LANES,), True, jnp.bool_)

            return refine(n, col_of, valid_of, t1, k1, 8)

        t, jcol = jax.lax.cond(qmax <= _CAP, fast_path, full_path)

        out_idx = lanes + jnp.full((_LANES,), s * _LANES, jnp.int32)
        plsc.store_scatter(tbuf, [out_idx], t)
        plsc.store_scatter(jbuf, [out_idx], jcol)
        return b0

    jax.lax.fori_loop(0, slabs, slab_body,
                      jnp.full((_LANES,), -1, jnp.int32))
    pltpu.sync_copy(tbuf, t_hbm.at[pl.ds(base_row, rows_per_w)])
    pltpu.sync_copy(jbuf, j_hbm.at[pl.ds(base_row, rows_per_w)])


def _centrality_kernel(mb_ref, out_ref):
    s = jnp.sum(mb_ref[...], axis=-1) / mb_ref.shape[-1]
    out_ref[...] = s.reshape(1, 1, -1)


def _loss_kernel(sim_ref, cent_ref, temp_ref, t_ref, j_ref, out_ref, *, rblk,
                 n):
    i = pl.program_id(0)
    sim = sim_ref[...]
    cent = cent_ref[...]
    temp = temp_ref[0, 0]
    tk = t_ref[...]
    jsel = j_ref[...]
    big = jnp.float32(_BIG)

    rows = i * rblk + jax.lax.broadcasted_iota(jnp.int32, (rblk, n), 0)
    cols = jax.lax.broadcasted_iota(jnp.int32, (rblk, n), 1)
    diag = rows == cols

    xi = jax.lax.bitcast_convert_type(sim, jnp.int32)
    key = jnp.where(xi < 0, xi ^ jnp.int32(0x7FFFFFFF), xi)
    key = jnp.where(diag, jnp.int32(-(2**31)), key)

    gt = key > tk
    eq = jnp.logical_and(key == tk, jnp.logical_not(diag))
    neighbor = jnp.logical_or(gt, jnp.logical_and(eq, cols <= jsel))
    extended = jnp.logical_or(neighbor, diag)
    comp = jnp.logical_not(extended)

    min_s = jnp.min(jnp.where(comp, sim, big), axis=1, keepdims=True)
    max_s = jnp.max(jnp.where(comp, sim, -big), axis=1, keepdims=True)
    min_c = jnp.min(jnp.where(comp, cent, big), axis=1, keepdims=True)
    max_c = jnp.max(jnp.where(comp, cent, -big), axis=1, keepdims=True)

    norm_s = (sim - min_s) / (max_s - min_s)
    norm_c = (cent - min_c) / (max_c - min_c)
    adj = jnp.where(neighbor, norm_s - norm_c, -big) * temp

    amax = jnp.max(adj, axis=1, keepdims=True)
    e = jnp.exp(adj - amax)
    w = e / jnp.sum(e, axis=1, keepdims=True)
    w = jnp.where(neighbor, w, 0.0)
    w = jnp.where(diag, 1.0, w)

    msim = jnp.where(extended, sim, -big)
    lmax = jnp.max(msim, axis=1, keepdims=True)
    lse = lmax + jnp.log(jnp.sum(jnp.exp(msim - lmax), axis=1, keepdims=True))
    lp = msim - lse

    numer = jnp.sum(w * lp, axis=1)
    denom = jnp.sum(w, axis=1)
    row_loss = -numer / denom

    @pl.when(i == 0)
    def _():
        out_ref[...] = jnp.zeros_like(out_ref)

    out_ref[...] += (jnp.sum(row_loss) / n).reshape(1, 1)


def kernel(similarity_matrix, memory_bank_matrix, num_neighbors, temperature):
    n = similarity_matrix.shape[0]
    rows_per_w = n // _NWORKERS

    karr = jnp.full((_LANES,), num_neighbors, jnp.int32)
    mesh = plsc.VectorSubcoreMesh(core_axis_name="c", subcore_axis_name="s")
    tsel, jsel = pl.kernel(
        functools.partial(_sc_select_kernel, n=n),
        mesh=mesh,
        compiler_params=pltpu.CompilerParams(needs_layout_passes=False),
        out_type=[
            jax.ShapeDtypeStruct((n,), jnp.int32),
            jax.ShapeDtypeStruct((n,), jnp.int32),
        ],
        scratch_types=[
            pltpu.VMEM((_LANES * n,), jnp.float32),
            pltpu.VMEM((rows_per_w,), jnp.int32),
            pltpu.VMEM((rows_per_w,), jnp.int32),
            pltpu.VMEM((_LANES,), jnp.int32),
            pltpu.VMEM((8 * 256 * _LANES,), jnp.int32),
            pltpu.VMEM((_CAP * _LANES,), jnp.int32),
        ],
    )(similarity_matrix.reshape(-1), karr)

    cblk = 256
    cent = pl.pallas_call(
        _centrality_kernel,
        grid=(n // cblk,),
        in_specs=[pl.BlockSpec((cblk, n), lambda i: (i, 0))],
        out_specs=pl.BlockSpec((1, 1, cblk), lambda i: (i, 0, 0)),
        out_shape=jax.ShapeDtypeStruct((n // cblk, 1, cblk), jnp.float32),
    )(memory_bank_matrix)
    cent = cent.reshape(1, n)

    rblk = 256
    loss = pl.pallas_call(
        functools.partial(_loss_kernel, rblk=rblk, n=n),
        grid=(n // rblk,),
        in_specs=[
            pl.BlockSpec((rblk, n), lambda i: (i, 0)),
            pl.BlockSpec((1, n), lambda i: (0, 0)),
            pl.BlockSpec((1, 1), lambda i: (0, 0)),
            pl.BlockSpec((rblk, 1), lambda i: (i, 0)),
            pl.BlockSpec((rblk, 1), lambda i: (i, 0)),
        ],
        out_specs=pl.BlockSpec((1, 1), lambda i: (0, 0)),
        out_shape=jax.ShapeDtypeStruct((1, 1), jnp.float32),
    )(
        similarity_matrix,
        cent,
        jnp.asarray(temperature, jnp.float32).reshape(1, 1),
        tsel.reshape(n, 1),
        jsel.reshape(n, 1),
    )
    return loss[0, 0]


# speedup vs baseline: 1.3881x; 1.0008x over previous
import functools

import jax
import jax.numpy as jnp
from jax.experimental import pallas as pl
from jax.experimental.pallas import tpu as pltpu
from jax.experimental.pallas import tpu_sc as plsc

_BIG = 9000000000000000.0
_LANES = 16
_NWORKERS = 32


_CAP = 1024


def _sc_select_kernel(sim_hbm, k_hbm, t_hbm, j_hbm, simbuf, tbuf, jbuf, kvbuf,
                      hist, candbuf, *, n):
    rows_per_w = n // _NWORKERS
    slabs = rows_per_w // _LANES
    cid = jax.lax.axis_index("c")
    sid = jax.lax.axis_index("s")
    wid = sid * 2 + cid
    base_row = wid * rows_per_w

    pltpu.sync_copy(k_hbm, kvbuf)
    kvec = kvbuf[...]
    lanes = jax.lax.iota(jnp.int32, _LANES)
    ones = jnp.ones((_LANES,), jnp.int32)
    zeros = jnp.zeros((_LANES,), jnp.int32)

    lanes_scaled = lanes * n
    neginf = jnp.full((_LANES,), -jnp.inf, jnp.float32)

    @plsc.parallel_loop(0, _CAP, unroll=8)
    def _(i):
        candbuf[pl.ds(i * _LANES, _LANES)] = zeros

    def slab_body(s, b_pred):
        row0 = base_row + s * _LANES
        pltpu.sync_copy(sim_hbm.at[pl.ds(row0 * n, _LANES * n)], simbuf)
        rowvec = lanes + jnp.full((_LANES,), row0, jnp.int32)
        plsc.store_scatter(simbuf, [lanes_scaled + rowvec], neginf)

        def key_gather(colv):
            v = plsc.load_gather(simbuf, [lanes_scaled + colv])
            xi = plsc.bitcast(v, jnp.int32)
            return jnp.where(xi < 0, xi ^ jnp.int32(0x7FFFFFFF), xi)

        def key_at(c):
            return key_gather(jnp.full((_LANES,), c, jnp.int32))

        def zero_hist(ncopy=1):
            @plsc.parallel_loop(0, 256 * ncopy, unroll=8)
            def _(b):
                hist[pl.ds(b * _LANES, _LANES)] = zeros

        def scan_level(k_cur, ncopy=1):
            def sbody(i, carry):
                cum, b0, above, hsel = carry
                b = 255 - i
                h = hist[pl.ds(b * _LANES, _LANES)]
                for p in range(1, ncopy):
                    h = h + hist[pl.ds((p * 256 + b) * _LANES, _LANES)]
                newcum = cum + h
                found = jnp.logical_and(cum < k_cur, newcum >= k_cur)
                bs = jnp.full((_LANES,), b, jnp.int32)
                b0 = jnp.where(found, bs, b0)
                above = jnp.where(found, cum, above)
                hsel = jnp.where(found, h, hsel)
                return newcum, b0, above, hsel

            _, b0, above, hsel = plsc.parallel_loop(
                0, 256, unroll=4, carry=(zeros, zeros, zeros, zeros))(sbody)
            return b0, above, hsel

        zero_hist()
        capv = jnp.full((_LANES,), _CAP, jnp.int32)

        def p1_body(c, cnt):
            key = key_at(c)
            bin_ = (key >> 24) + 128
            plsc.addupdate_scatter(hist, [bin_ * _LANES + lanes], ones)
            am = jnp.logical_and(bin_ == b_pred, cnt < capv)
            plsc.store_scatter(candbuf, [cnt * _LANES + lanes],
                               jnp.full((_LANES,), c, jnp.int32), mask=am)
            return cnt + jnp.where(am, 1, 0)

        plsc.parallel_loop(0, n, unroll=16, carry=zeros)(p1_body)

        b0, above, q = scan_level(kvec)
        t1 = b0 - 128
        k1 = kvec - above
        qmax = jnp.max(q)
        pred_ok = jnp.all(b0 == b_pred)

        def refine(loop_hi, col_of, valid_of, t_in, k_in, unroll):
            t = t_in
            k_cur = k_in
            for shift in (16, 8, 0):
                zero_hist()

                def lvl_body(c, shift=shift, t=t):
                    key = key_gather(col_of(c))
                    x = key >> shift
                    match = jnp.logical_and((x >> 8) == t, valid_of(c))
                    plsc.addupdate_scatter(hist,
                                           [(x & 255) * _LANES + lanes],
                                           ones, mask=match)

                plsc.parallel_loop(0, loop_hi, unroll=unroll)(lvl_body)
                b0, above, _ = scan_level(k_cur)
                t = (t << 8) | b0
                k_cur = k_cur - above

            def tie_body(c, carry):
                cnt, jcol = carry
                colv = col_of(c)
                key = key_gather(colv)
                e = jnp.logical_and(key == t, valid_of(c))
                hit = jnp.logical_and(e, cnt == k_cur - 1)
                jcol = jnp.where(hit, colv, jcol)
                cnt = cnt + jnp.where(e, 1, 0)
                return cnt, jcol

            _, jcol = plsc.parallel_loop(0, loop_hi, unroll=unroll,
                                         carry=(zeros, zeros))(tie_body)
            return t, jcol

        def fast_path():
            def rebuild():
                def comp_body(c, cnt):
                    key = key_at(c)
                    match = ((key >> 24) + 128) == b0
                    plsc.store_scatter(candbuf, [cnt * _LANES + lanes],
                                       jnp.full((_LANES,), c, jnp.int32),
                                       mask=match)
                    return cnt + jnp.where(match, 1, 0)

                plsc.parallel_loop(0, n, unroll=16, carry=zeros)(comp_body)
                return jnp.int32(0)

            jax.lax.cond(pred_ok, lambda: jnp.int32(0), rebuild)

            def col_of(c):
                return plsc.load_gather(
                    candbuf,
                    [jnp.full((_LANES,), c * _LANES, jnp.int32) + lanes])

            def valid_of(c):
                return jnp.full((_LANES,), c, jnp.int32) < q

            return refine(qmax, col_of, valid_of, t1, k1, 4)

        def full_path():
            def col_of(c):
                return jnp.full((_LANES,), c, jnp.int32)

            def valid_of(c):
                return jnp.full((_LANES,), True, jnp.bool_)

            return refine(n, col_of, valid_of, t1, k1, 8)

        t, jcol = jax.lax.cond(qmax <= _CAP, fast_path, full_path)

        out_idx = lanes + jnp.full((_LANES,), s * _LANES, jnp.int32)
        plsc.store_scatter(tbuf, [out_idx], t)
        plsc.store_scatter(jbuf, [out_idx], jcol)
        return b0

    jax.lax.fori_loop(0, slabs, slab_body,
                      jnp.full((_LANES,), -1, jnp.int32))
    pltpu.sync_copy(tbuf, t_hbm.at[pl.ds(base_row, rows_per_w)])
    pltpu.sync_copy(jbuf, j_hbm.at[pl.ds(base_row, rows_per_w)])


def _centrality_kernel(mb_ref, out_ref):
    s = jnp.sum(mb_ref[...], axis=-1) / mb_ref.shape[-1]
    out_ref[...] = s.reshape(1, 1, -1)


def _loss_kernel(sim_ref, cent_ref, temp_ref, t_ref, j_ref, out_ref, *, rblk,
                 n):
    i = pl.program_id(0)
    sim = sim_ref[...]
    cent = cent_ref[...]
    temp = temp_ref[0, 0]
    tk = t_ref[...]
    jsel = j_ref[...]
    big = jnp.float32(_BIG)

    rows = i * rblk + jax.lax.broadcasted_iota(jnp.int32, (rblk, n), 0)
    cols = jax.lax.broadcasted_iota(jnp.int32, (rblk, n), 1)
    diag = rows == cols

    xi = jax.lax.bitcast_convert_type(sim, jnp.int32)
    key = jnp.where(xi < 0, xi ^ jnp.int32(0x7FFFFFFF), xi)
    key = jnp.where(diag, jnp.int32(-(2**31)), key)

    gt = key > tk
    eq = jnp.logical_and(key == tk, jnp.logical_not(diag))
    neighbor = jnp.logical_or(gt, jnp.logical_and(eq, cols <= jsel))
    extended = jnp.logical_or(neighbor, diag)
    comp = jnp.logical_not(extended)

    min_s = jnp.min(jnp.where(comp, sim, big), axis=1, keepdims=True)
    max_s = jnp.max(jnp.where(comp, sim, -big), axis=1, keepdims=True)
    min_c = jnp.min(jnp.where(comp, cent, big), axis=1, keepdims=True)
    max_c = jnp.max(jnp.where(comp, cent, -big), axis=1, keepdims=True)

    norm_s = (sim - min_s) / (max_s - min_s)
    norm_c = (cent - min_c) / (max_c - min_c)
    adj = jnp.where(neighbor, norm_s - norm_c, -big) * temp

    amax = jnp.max(adj, axis=1, keepdims=True)
    e = jnp.exp(adj - amax)
    w = e / jnp.sum(e, axis=1, keepdims=True)
    w = jnp.where(neighbor, w, 0.0)
    w = jnp.where(diag, 1.0, w)

    msim = jnp.where(extended, sim, -big)
    lmax = jnp.max(msim, axis=1, keepdims=True)
    lse = lmax + jnp.log(jnp.sum(jnp.exp(msim - lmax), axis=1, keepdims=True))
    lp = msim - lse

    numer = jnp.sum(w * lp, axis=1)
    denom = jnp.sum(w, axis=1)
    row_loss = -numer / denom

    @pl.when(i == 0)
    def _():
        out_ref[...] = jnp.zeros_like(out_ref)

    out_ref[...] += (jnp.sum(row_loss) / n).reshape(1, 1)


def kernel(similarity_matrix, memory_bank_matrix, num_neighbors, temperature):
    n = similarity_matrix.shape[0]
    rows_per_w = n // _NWORKERS

    karr = jnp.full((_LANES,), num_neighbors, jnp.int32)
    mesh = plsc.VectorSubcoreMesh(core_axis_name="c", subcore_axis_name="s")
    tsel, jsel = pl.kernel(
        functools.partial(_sc_select_kernel, n=n),
        mesh=mesh,
        compiler_params=pltpu.CompilerParams(needs_layout_passes=False),
        out_type=[
            jax.ShapeDtypeStruct((n,), jnp.int32),
            jax.ShapeDtypeStruct((n,), jnp.int32),
        ],
        scratch_types=[
            pltpu.VMEM((_LANES * n,), jnp.float32),
            pltpu.VMEM((rows_per_w,), jnp.int32),
            pltpu.VMEM((rows_per_w,), jnp.int32),
            pltpu.VMEM((_LANES,), jnp.int32),
            pltpu.VMEM((256 * _LANES,), jnp.int32),
            pltpu.VMEM((_CAP * _LANES,), jnp.int32),
        ],
    )(similarity_matrix.reshape(-1), karr)

    cblk = 256
    cent = pl.pallas_call(
        _centrality_kernel,
        grid=(n // cblk,),
        in_specs=[pl.BlockSpec((cblk, n), lambda i: (i, 0))],
        out_specs=pl.BlockSpec((1, 1, cblk), lambda i: (i, 0, 0)),
        out_shape=jax.ShapeDtypeStruct((n // cblk, 1, cblk), jnp.float32),
    )(memory_bank_matrix)
    cent = cent.reshape(1, n)

    rblk = 256
    loss = pl.pallas_call(
        functools.partial(_loss_kernel, rblk=rblk, n=n),
        grid=(n // rblk,),
        in_specs=[
            pl.BlockSpec((rblk, n), lambda i: (i, 0)),
            pl.BlockSpec((1, n), lambda i: (0, 0)),
            pl.BlockSpec((1, 1), lambda i: (0, 0)),
            pl.BlockSpec((rblk, 1), lambda i: (i, 0)),
            pl.BlockSpec((rblk, 1), lambda i: (i, 0)),
        ],
        out_specs=pl.BlockSpec((1, 1), lambda i: (0, 0)),
        out_shape=jax.ShapeDtypeStruct((1, 1), jnp.float32),
    )(
        similarity_matrix,
        cent,
        jnp.asarray(temperature, jnp.float32).reshape(1, 1),
        tsel.reshape(n, 1),
        jsel.reshape(n, 1),
    )
    return loss[0, 0]
